# Initial kernel scaffold; baseline (speedup 1.0000x reference)
import jax
import jax.numpy as jnp
from jax.experimental import pallas as pl


def kernel(x, edge_index, Wq, bq, Wk, bk, Wv, bv, Wskip, bskip, g1, beta1, g2, beta2, W1, bf1, W2, bf2, res_scale):
    raise NotImplementedError("write your pallas kernel here")


# speedup vs baseline: 21.2520x; 21.2520x over previous
import functools

import jax
import jax.numpy as jnp
from jax import lax
from jax.experimental import pallas as pl
from jax.experimental.pallas import tpu as pltpu
from jax.experimental.pallas import tpu_sc as plsc

N = 10000
E = 160000
D = 256
H = 4
DH = 64
PW = 128
BN = 1000
BE = 64
NSUB = 16
NP = 10240
SLP = 2 * NP
NZSUB = 10
SR = SLP // PW
_EPS = 1e-5


def _pre_body(x_ref, wq_ref, bq_ref, wk_ref, bk_ref, wv_ref, bv_ref,
              ws_ref, bs_ref, g1_ref, be1_ref,
              qP_ref, kP_ref, vP_ref, xs_ref):
    xb = x_ref[...]
    mu = jnp.mean(xb, axis=-1, keepdims=True)
    var = jnp.mean((xb - mu) ** 2, axis=-1, keepdims=True)
    xn = (xb - mu) * lax.rsqrt(var + _EPS) * g1_ref[...] + be1_ref[...]
    q = jnp.dot(xn, wq_ref[...], preferred_element_type=jnp.float32) + bq_ref[...]
    k = jnp.dot(xn, wk_ref[...], preferred_element_type=jnp.float32) + bk_ref[...]
    v = jnp.dot(xn, wv_ref[...], preferred_element_type=jnp.float32) + bv_ref[...]
    xs_ref[...] = jnp.dot(xn, ws_ref[...], preferred_element_type=jnp.float32) + bs_ref[...]
    scale = jnp.float32(DH ** -0.5)
    for p in range(2):
        sl = slice(p * PW, (p + 1) * PW)
        qP_ref[p] = q[:, sl] * scale
        kP_ref[p] = k[:, sl]
        vP_ref[p] = v[:, sl]


def _pre(x, Wq, bq, Wk, bk, Wv, bv, Wskip, bskip, g1, beta1):
    full = lambda shp: pl.BlockSpec(shp, lambda i: (0,) * len(shp))
    return pl.pallas_call(
        _pre_body,
        grid=(N // BN,),
        in_specs=[
            pl.BlockSpec((BN, D), lambda i: (i, 0)),
            full((D, D)), full((1, D)),
            full((D, D)), full((1, D)),
            full((D, D)), full((1, D)),
            full((D, D)), full((1, D)),
            full((1, D)), full((1, D)),
        ],
        out_specs=[
            pl.BlockSpec((2, BN, PW), lambda i: (0, i, 0)),
            pl.BlockSpec((2, BN, PW), lambda i: (0, i, 0)),
            pl.BlockSpec((2, BN, PW), lambda i: (0, i, 0)),
            pl.BlockSpec((BN, D), lambda i: (i, 0)),
        ],
        out_shape=[
            jax.ShapeDtypeStruct((2, N, PW), jnp.float32),
            jax.ShapeDtypeStruct((2, N, PW), jnp.float32),
            jax.ShapeDtypeStruct((2, N, PW), jnp.float32),
            jax.ShapeDtypeStruct((N, D), jnp.float32),
        ],
    )(x, Wq, bq.reshape(1, D), Wk, bk.reshape(1, D), Wv, bv.reshape(1, D),
      Wskip, bskip.reshape(1, D), g1.reshape(1, D), beta1.reshape(1, D))


_TAKE_DNUMS = lax.GatherDimensionNumbers(
    offset_dims=(), collapsed_slice_dims=(0,), start_index_map=(0,))


def _take16(v, idx):
    return lax.gather(v, idx[:, None], _TAKE_DNUMS, (1,),
                      mode=lax.GatherScatterMode.PROMISE_IN_BOUNDS)


def _sc_edge(qP, kP, vP, src, dst):
    mesh = plsc.VectorSubcoreMesh(core_axis_name="c", subcore_axis_name="s")

    @functools.partial(
        pl.kernel,
        mesh=mesh,
        out_type=[
            jax.ShapeDtypeStruct((2 * NP, PW), jnp.float32),
            jax.ShapeDtypeStruct((2 * NSUB * SR, PW), jnp.float32),
        ],
        scratch_types=[
            pltpu.VMEM((BE,), jnp.int32),
            pltpu.VMEM((BE,), jnp.int32),
            pltpu.VMEM((BE,), jnp.int32),
            pltpu.VMEM((BE,), jnp.int32),
            pltpu.VMEM((BE, PW), jnp.float32),
            pltpu.VMEM((BE, PW), jnp.float32),
            pltpu.VMEM((BE, PW), jnp.float32),
            pltpu.VMEM((SR, PW), jnp.float32),
            pltpu.VMEM_SHARED((NP, PW), jnp.float32),
            pltpu.SemaphoreType.DMA,
            pltpu.SemaphoreType.DMA,
            pltpu.SemaphoreType.DMA,
        ],
    )
    def body(qP_h, kP_h, vP_h, src_h, dst_h, out_h, sview_h,
             srcb, dstb, idxq, idxk, qrows, krows, vrows,
             s_local, acc, sem1, sem2, sem3):
        c = lax.axis_index("c")
        s = lax.axis_index("s")
        cN = c * N
        lanes = lax.iota(jnp.int32, 16)
        zero16 = jnp.zeros((16,), jnp.float32)

        def _zero_row(i, carry):
            for t in range(PW // 16):
                qrows[i, pl.ds(t * 16, 16)] = zero16
            return carry
        lax.fori_loop(0, BE, _zero_row, 0)

        def _zero_s(i, carry):
            for t in range(PW // 16):
                s_local[i, pl.ds(t * 16, 16)] = zero16
            return carry
        lax.fori_loop(0, SR, _zero_s, 0)

        stripe = NP // NZSUB
        row0 = s * stripe

        @pl.when(s < NZSUB)
        def _zero_acc():
            def _zcp(t, carry):
                pltpu.sync_copy(qrows, acc.at[pl.ds(row0 + t * BE, BE)])
                return carry
            lax.fori_loop(0, stripe // BE, _zcp, 0)
        plsc.subcore_barrier()

        def _blk(i, carry):
            base = (s + i * NSUB) * BE
            pltpu.sync_copy(src_h.at[pl.ds(base, BE)], srcb)
            pltpu.sync_copy(dst_h.at[pl.ds(base, BE)], dstb)
            for t in range(BE // 16):
                sl = pl.ds(t * 16, 16)
                idxk[sl] = srcb[sl] + cN
                idxq[sl] = dstb[sl] + cN
            cq = pltpu.async_copy(qP_h.at[idxq], qrows, sem1)
            ck = pltpu.async_copy(kP_h.at[idxk], krows, sem2)
            cv = pltpu.async_copy(vP_h.at[idxk], vrows, sem3)
            cq.wait()
            ck.wait()
            cv.wait()

            def _group(g, gcarry):
                jbase = g * 16
                dst16 = dstb[pl.ds(jbase, 16)]
                for j2 in range(16):
                    j = jbase + j2
                    d0 = qrows[j, pl.ds(0, 16)] * krows[j, pl.ds(0, 16)]
                    d1 = qrows[j, pl.ds(16, 16)] * krows[j, pl.ds(16, 16)]
                    d2 = qrows[j, pl.ds(32, 16)] * krows[j, pl.ds(32, 16)]
                    d3 = qrows[j, pl.ds(48, 16)] * krows[j, pl.ds(48, 16)]
                    da = (d0 + d1) + (d2 + d3)
                    d4 = qrows[j, pl.ds(64, 16)] * krows[j, pl.ds(64, 16)]
                    d5 = qrows[j, pl.ds(80, 16)] * krows[j, pl.ds(80, 16)]
                    d6 = qrows[j, pl.ds(96, 16)] * krows[j, pl.ds(96, 16)]
                    d7 = qrows[j, pl.ds(112, 16)] * krows[j, pl.ds(112, 16)]
                    db = (d4 + d5) + (d6 + d7)
                    for sh in (8, 4, 2, 1):
                        da = da + _take16(da, lanes ^ sh)
                        db = db + _take16(db, lanes ^ sh)
                    ev0 = jnp.exp(da)
                    ev1 = jnp.exp(db)
                    for t in range(4):
                        sl = pl.ds(t * 16, 16)
                        vrows[j, sl] = vrows[j, sl] * ev0
                    for t in range(4, 8):
                        sl = pl.ds(t * 16, 16)
                        vrows[j, sl] = vrows[j, sl] * ev1
                    sidx = dst16[j2] * 2
                    srow = sidx >> 7
                    roff = ((sidx & 127) >> 4) << 4
                    wlane = sidx & 15
                    win = pl.ds(roff, 16)
                    contrib = (jnp.where(lanes == wlane, ev0, zero16)
                               + jnp.where(lanes == wlane + 1, ev1, zero16))
                    s_local[srow, win] = s_local[srow, win] + contrib
                return gcarry
            lax.fori_loop(0, BE // 16, _group, 0)
            pltpu.sync_copy(vrows, acc.at[dstb], add=True)
            return carry
        nb = (E // BE - s + NSUB - 1) // NSUB
        lax.fori_loop(0, nb, _blk, 0)

        pltpu.sync_copy(s_local, sview_h.at[pl.ds((c * NSUB + s) * SR, SR)])
        plsc.subcore_barrier()

        @pl.when(s < NZSUB)
        def _finish():
            srow0 = s * 16
            pltpu.sync_copy(sview_h.at[pl.ds(c * NSUB * SR + srow0, 16)],
                            vrows.at[pl.ds(16, 16)])

            def _radd(r, carry):
                pltpu.sync_copy(
                    sview_h.at[pl.ds((c * NSUB + r) * SR + srow0, 16)],
                    vrows.at[pl.ds(0, 16)])

                def _vadd(i, c2):
                    for t in range(PW // 16):
                        sl = pl.ds(t * 16, 16)
                        vrows[16 + i, sl] = vrows[16 + i, sl] + vrows[i, sl]
                    return c2
                lax.fori_loop(0, 16, _vadd, 0)
                return carry
            lax.fori_loop(1, NSUB, _radd, 0)

            one16 = jnp.full((16,), 1.0, jnp.float32)
            tiny16 = jnp.full((16,), 1e-30, jnp.float32)

            def _rinv(i, carry):
                for t in range(PW // 16):
                    sl = pl.ds(t * 16, 16)
                    vrows[16 + i, sl] = one16 / jnp.maximum(vrows[16 + i, sl], tiny16)
                return carry
            lax.fori_loop(0, 16, _rinv, 0)

            def _ocp(t, carry):
                pltpu.sync_copy(acc.at[pl.ds(row0 + t * BE, BE)], qrows)

                def _nrow(j, c2):
                    rel = (t * BE + j) * 2
                    rrow = 16 + (rel >> 7)
                    roff = ((rel & 127) >> 4) << 4
                    wlane = rel & 15
                    rv = vrows[rrow, pl.ds(roff, 16)]
                    r0 = _take16(rv, lanes * 0 + wlane)
                    r1 = _take16(rv, lanes * 0 + wlane + 1)
                    for t4 in range(4):
                        sl = pl.ds(t4 * 16, 16)
                        qrows[j, sl] = qrows[j, sl] * r0
                    for t4 in range(4, 8):
                        sl = pl.ds(t4 * 16, 16)
                        qrows[j, sl] = qrows[j, sl] * r1
                    return c2
                lax.fori_loop(0, BE, _nrow, 0)
                pltpu.sync_copy(qrows, out_h.at[pl.ds(c * NP + row0 + t * BE, BE)])
                return carry
            lax.fori_loop(0, stripe // BE, _ocp, 0)

    return body(qP, kP, vP, src, dst)


def _post_body(x_ref, agg_ref, xs_ref, g2_ref, b2_ref, w1_ref, bf1_ref,
               w2_ref, bf2_ref, rs_ref, o_ref):
    rs = rs_ref[0, 0]
    agg = jnp.concatenate([agg_ref[0], agg_ref[1]], axis=1)
    out = agg + xs_ref[...]
    x1 = x_ref[...] + rs * out
    mu = jnp.mean(x1, axis=-1, keepdims=True)
    var = jnp.mean((x1 - mu) ** 2, axis=-1, keepdims=True)
    hh = (x1 - mu) * lax.rsqrt(var + _EPS) * g2_ref[...] + b2_ref[...]
    y = jnp.dot(hh, w1_ref[...], preferred_element_type=jnp.float32) + bf1_ref[...]
    f = y * jnp.float32(0.5) * (jnp.float32(1.0) + lax.erf(y * jnp.float32(0.7071067811865476)))
    f = jnp.dot(f, w2_ref[...], preferred_element_type=jnp.float32) + bf2_ref[...]
    o_ref[...] = x1 + rs * f


def _post(x, agg, xskip, g2, beta2, W1, bf1, W2, bf2, res_scale):
    full = lambda shp: pl.BlockSpec(shp, lambda i: (0,) * len(shp))
    return pl.pallas_call(
        _post_body,
        grid=(N // BN,),
        in_specs=[
            pl.BlockSpec((BN, D), lambda i: (i, 0)),
            pl.BlockSpec((2, BN, PW), lambda i: (0, i, 0)),
            pl.BlockSpec((BN, D), lambda i: (i, 0)),
            full((1, D)), full((1, D)),
            full((D, D)), full((1, D)),
            full((D, D)), full((1, D)),
            pl.BlockSpec(memory_space=pltpu.SMEM),
        ],
        out_specs=pl.BlockSpec((BN, D), lambda i: (i, 0)),
        out_shape=jax.ShapeDtypeStruct((N, D), jnp.float32),
    )(x, agg, xskip, g2.reshape(1, D), beta2.reshape(1, D),
      W1, bf1.reshape(1, D), W2, bf2.reshape(1, D), res_scale.reshape(1, 1))


def kernel(x, edge_index, Wq, bq, Wk, bk, Wv, bv, Wskip, bskip,
           g1, beta1, g2, beta2, W1, bf1, W2, bf2, res_scale):
    src = edge_index[0]
    dst = edge_index[1]
    qP, kP, vP, xskip = _pre(x, Wq, bq, Wk, bk, Wv, bv, Wskip, bskip, g1, beta1)
    aggP, _ = _sc_edge(qP.reshape(2 * N, PW), kP.reshape(2 * N, PW),
                       vP.reshape(2 * N, PW), src, dst)
    return _post(x, aggP.reshape(2, NP, PW)[:, :N, :], xskip,
                 g2, beta2, W1, bf1, W2, bf2, res_scale)


# speedup vs baseline: 26.3571x; 1.2402x over previous
import functools

import jax
import jax.numpy as jnp
from jax import lax
from jax.experimental import pallas as pl
from jax.experimental.pallas import tpu as pltpu
from jax.experimental.pallas import tpu_sc as plsc

N = 10000
E = 160000
D = 256
H = 4
DH = 64
PW = 128
BN = 1000
BE = 32
NSUB = 16
NP = 10240
SLP = 2 * NP
NZSUB = 16
SR = SLP // PW
_EPS = 1e-5


def _pre_body(x_ref, wq_ref, bq_ref, wk_ref, bk_ref, wv_ref, bv_ref,
              ws_ref, bs_ref, g1_ref, be1_ref,
              qP_ref, kP_ref, vP_ref, xs_ref):
    xb = x_ref[...]
    mu = jnp.mean(xb, axis=-1, keepdims=True)
    var = jnp.mean((xb - mu) ** 2, axis=-1, keepdims=True)
    xn = (xb - mu) * lax.rsqrt(var + _EPS) * g1_ref[...] + be1_ref[...]
    q = jnp.dot(xn, wq_ref[...], preferred_element_type=jnp.float32) + bq_ref[...]
    k = jnp.dot(xn, wk_ref[...], preferred_element_type=jnp.float32) + bk_ref[...]
    v = jnp.dot(xn, wv_ref[...], preferred_element_type=jnp.float32) + bv_ref[...]
    xs_ref[...] = jnp.dot(xn, ws_ref[...], preferred_element_type=jnp.float32) + bs_ref[...]
    scale = jnp.float32(DH ** -0.5)
    for p in range(2):
        sl = slice(p * PW, (p + 1) * PW)
        qP_ref[p] = q[:, sl] * scale
        kP_ref[p] = k[:, sl]
        vP_ref[p] = v[:, sl]


def _pre(x, Wq, bq, Wk, bk, Wv, bv, Wskip, bskip, g1, beta1):
    full = lambda shp: pl.BlockSpec(shp, lambda i: (0,) * len(shp))
    return pl.pallas_call(
        _pre_body,
        grid=(N // BN,),
        in_specs=[
            pl.BlockSpec((BN, D), lambda i: (i, 0)),
            full((D, D)), full((1, D)),
            full((D, D)), full((1, D)),
            full((D, D)), full((1, D)),
            full((D, D)), full((1, D)),
            full((1, D)), full((1, D)),
        ],
        out_specs=[
            pl.BlockSpec((2, BN, PW), lambda i: (0, i, 0)),
            pl.BlockSpec((2, BN, PW), lambda i: (0, i, 0)),
            pl.BlockSpec((2, BN, PW), lambda i: (0, i, 0)),
            pl.BlockSpec((BN, D), lambda i: (i, 0)),
        ],
        out_shape=[
            jax.ShapeDtypeStruct((2, N, PW), jnp.float32),
            jax.ShapeDtypeStruct((2, N, PW), jnp.float32),
            jax.ShapeDtypeStruct((2, N, PW), jnp.float32),
            jax.ShapeDtypeStruct((N, D), jnp.float32),
        ],
    )(x, Wq, bq.reshape(1, D), Wk, bk.reshape(1, D), Wv, bv.reshape(1, D),
      Wskip, bskip.reshape(1, D), g1.reshape(1, D), beta1.reshape(1, D))


_TAKE_DNUMS = lax.GatherDimensionNumbers(
    offset_dims=(), collapsed_slice_dims=(0,), start_index_map=(0,))


def _take16(v, idx):
    return lax.gather(v, idx[:, None], _TAKE_DNUMS, (1,),
                      mode=lax.GatherScatterMode.PROMISE_IN_BOUNDS)


def _sc_edge(qP, kP, vP, src, dst):
    mesh = plsc.VectorSubcoreMesh(core_axis_name="c", subcore_axis_name="s")

    @functools.partial(
        pl.kernel,
        mesh=mesh,
        out_type=[
            jax.ShapeDtypeStruct((2 * NP, PW), jnp.float32),
            jax.ShapeDtypeStruct((2 * NSUB * SLP,), jnp.float32),
        ],
        scratch_types=[
            pltpu.VMEM((BE,), jnp.int32),
            pltpu.VMEM((BE,), jnp.int32),
            pltpu.VMEM((BE,), jnp.int32),
            pltpu.VMEM((BE,), jnp.int32),
            pltpu.VMEM((BE,), jnp.int32),
            pltpu.VMEM((BE,), jnp.int32),
            pltpu.VMEM((BE,), jnp.int32),
            pltpu.VMEM((BE,), jnp.int32),
            pltpu.VMEM((BE,), jnp.int32),
            pltpu.VMEM((BE,), jnp.int32),
            pltpu.VMEM((BE, PW), jnp.float32),
            pltpu.VMEM((BE, PW), jnp.float32),
            pltpu.VMEM((BE, PW), jnp.float32),
            pltpu.VMEM((BE, PW), jnp.float32),
            pltpu.VMEM((BE, PW), jnp.float32),
            pltpu.VMEM((BE, PW), jnp.float32),
            pltpu.VMEM((SLP,), jnp.float32),
            pltpu.VMEM_SHARED((NP, PW), jnp.float32),
            pltpu.SemaphoreType.DMA,
            pltpu.SemaphoreType.DMA,
            pltpu.SemaphoreType.DMA,
            pltpu.SemaphoreType.DMA,
            pltpu.SemaphoreType.DMA,
            pltpu.SemaphoreType.DMA,
            pltpu.SemaphoreType.DMA,
            pltpu.SemaphoreType.DMA,
        ],
    )
    def body(qP_h, kP_h, vP_h, src_h, dst_h, out_h, sview_h,
             srcbA, srcbB, dstbA, dstbB, idxqA, idxqB, idxkA, idxkB,
             idxaA, idxaB, qrA, qrB, krA, krB, vrA, vrB, s_local, acc,
             sem_i, sem_qA, sem_qB, sem_kA, sem_kB, sem_vA, sem_vB, sem_s):
        c = lax.axis_index("c")
        s = lax.axis_index("s")
        cN = c * N
        lanes = lax.iota(jnp.int32, 16)
        zero16 = jnp.zeros((16,), jnp.float32)

        setA = (srcbA, dstbA, idxqA, idxkA, idxaA, qrA, krA, vrA,
                sem_qA, sem_kA, sem_vA)
        setB = (srcbB, dstbB, idxqB, idxkB, idxaB, qrB, krB, vrB,
                sem_qB, sem_kB, sem_vB)

        def _zero_row(i, carry):
            for t in range(PW // 16):
                qrA[i, pl.ds(t * 16, 16)] = zero16
            return carry
        lax.fori_loop(0, BE, _zero_row, 0)

        def _zero_s(i, carry):
            s_local[pl.ds(i * 16, 16)] = zero16
            return carry
        lax.fori_loop(0, SLP // 16, _zero_s, 0)

        stripe = NP // NZSUB
        row0 = s * stripe

        @pl.when(s < NZSUB)
        def _zero_acc():
            def _zcp(t, carry):
                pltpu.sync_copy(qrA, acc.at[pl.ds(row0 + t * BE, BE)])
                return carry
            lax.fori_loop(0, stripe // BE, _zcp, 0)
        plsc.subcore_barrier()

        def gbase(j):
            return jnp.minimum((s + j * NSUB) * BE, E - BE)

        def load_idx(j, st):
            b = gbase(j)
            pltpu.async_copy(src_h.at[pl.ds(b, BE)], st[0], sem_i)
            pltpu.async_copy(dst_h.at[pl.ds(b, BE)], st[1], sem_i)

        def wait_idx(st):
            pltpu.make_async_copy(src_h.at[pl.ds(0, BE)], st[0], sem_i).wait()
            pltpu.make_async_copy(dst_h.at[pl.ds(0, BE)], st[1], sem_i).wait()

        def prep(st):
            for t in range(BE // 16):
                sl = pl.ds(t * 16, 16)
                st[3][sl] = st[0][sl] + cN
                st[2][sl] = st[1][sl] + cN
                st[4][sl] = st[1][sl]

        def gathers(st):
            pltpu.async_copy(qP_h.at[st[2]], st[5], st[8])
            pltpu.async_copy(kP_h.at[st[3]], st[6], st[9])
            pltpu.async_copy(vP_h.at[st[3]], st[7], st[10])

        def wait_gathers(st):
            pltpu.make_async_copy(qP_h.at[st[2]], st[5], st[8]).wait()
            pltpu.make_async_copy(kP_h.at[st[3]], st[6], st[9]).wait()
            pltpu.make_async_copy(vP_h.at[st[3]], st[7], st[10]).wait()

        def drain_scatter(st):
            pltpu.make_async_copy(st[7], acc.at[st[4]], sem_s).wait()

        def compute(st):
            qr, kr, vr, idxa_t = st[5], st[6], st[7], st[4]

            def _group(g, gcarry):
                jbase = g * 16
                dst16 = idxa_t[pl.ds(jbase, 16)]
                for j2 in range(16):
                    j = jbase + j2
                    d0 = qr[j, pl.ds(0, 16)] * kr[j, pl.ds(0, 16)]
                    d1 = qr[j, pl.ds(16, 16)] * kr[j, pl.ds(16, 16)]
                    d2 = qr[j, pl.ds(32, 16)] * kr[j, pl.ds(32, 16)]
                    d3 = qr[j, pl.ds(48, 16)] * kr[j, pl.ds(48, 16)]
                    da = (d0 + d1) + (d2 + d3)
                    d4 = qr[j, pl.ds(64, 16)] * kr[j, pl.ds(64, 16)]
                    d5 = qr[j, pl.ds(80, 16)] * kr[j, pl.ds(80, 16)]
                    d6 = qr[j, pl.ds(96, 16)] * kr[j, pl.ds(96, 16)]
                    d7 = qr[j, pl.ds(112, 16)] * kr[j, pl.ds(112, 16)]
                    db = (d4 + d5) + (d6 + d7)
                    for sh in (8, 4, 2, 1):
                        da = da + _take16(da, lanes ^ sh)
                        db = db + _take16(db, lanes ^ sh)
                    ev0 = jnp.exp(da)
                    ev1 = jnp.exp(db)
                    for t in range(4):
                        sl = pl.ds(t * 16, 16)
                        vr[j, sl] = vr[j, sl] * ev0
                    for t in range(4, 8):
                        sl = pl.ds(t * 16, 16)
                        vr[j, sl] = vr[j, sl] * ev1
                    sidx = dst16[j2] * 2
                    woff = (sidx >> 4) << 4
                    wlane = sidx & 15
                    win = pl.ds(woff, 16)
                    contrib = (jnp.where(lanes == wlane, ev0, zero16)
                               + jnp.where(lanes == wlane + 1, ev1, zero16))
                    s_local[win] = s_local[win] + contrib
                return gcarry
            lax.fori_loop(0, BE // 16, _group, 0)

        def scatter(st):
            pltpu.async_copy(st[7], acc.at[st[4]], sem_s, add=True)

        b0 = gbase(0)
        pltpu.sync_copy(src_h.at[pl.ds(b0, BE)], srcbA)
        pltpu.sync_copy(dst_h.at[pl.ds(b0, BE)], dstbA)
        prep(setA)
        gathers(setA)
        load_idx(1, setB)

        NB2 = E // BE // NSUB // 2

        def _pair(i2, carry):
            wait_idx(setB)

            @pl.when(i2 > 0)
            def _():
                drain_scatter(setB)
            prep(setB)
            gathers(setB)
            load_idx(2 * i2 + 2, setA)
            wait_gathers(setA)
            compute(setA)
            scatter(setA)
            wait_idx(setA)
            drain_scatter(setA)
            prep(setA)
            gathers(setA)
            load_idx(2 * i2 + 3, setB)
            wait_gathers(setB)
            compute(setB)
            scatter(setB)
            return carry
        lax.fori_loop(0, NB2, _pair, 0)

        wait_idx(setB)
        drain_scatter(setB)
        wait_gathers(setA)

        @pl.when(s < 8)
        def _tail():
            compute(setA)
            pltpu.sync_copy(vrA, acc.at[idxaA], add=True)

        pltpu.sync_copy(s_local, sview_h.at[pl.ds((c * NSUB + s) * SLP, SLP)])
        plsc.subcore_barrier()

        @pl.when(s < NZSUB)
        def _finish():
            soff = s * 1280
            pltpu.sync_copy(sview_h.at[pl.ds(c * NSUB * SLP + soff, 1280)],
                            s_local.at[pl.ds(2048, 1280)])

            def _radd(r, carry):
                pltpu.sync_copy(
                    sview_h.at[pl.ds((c * NSUB + r) * SLP + soff, 1280)],
                    s_local.at[pl.ds(0, 1280)])

                def _vadd(i, c2):
                    sl = pl.ds(2048 + i * 16, 16)
                    s_local[sl] = s_local[sl] + s_local[pl.ds(i * 16, 16)]
                    return c2
                lax.fori_loop(0, 80, _vadd, 0)
                return carry
            lax.fori_loop(1, NSUB, _radd, 0)

            one16 = jnp.full((16,), 1.0, jnp.float32)
            tiny16 = jnp.full((16,), 1e-30, jnp.float32)

            def _rinv(i, carry):
                sl = pl.ds(2048 + i * 16, 16)
                s_local[sl] = one16 / jnp.maximum(s_local[sl], tiny16)
                return carry
            lax.fori_loop(0, 80, _rinv, 0)

            def _ocp(t, carry):
                pltpu.sync_copy(acc.at[pl.ds(row0 + t * BE, BE)], qrA)

                def _nrow(j, c2):
                    rel = (t * BE + j) * 2
                    woff = (rel >> 4) << 4
                    wlane = rel & 15
                    rv = s_local[pl.ds(2048 + woff, 16)]
                    r0 = _take16(rv, lanes * 0 + wlane)
                    r1 = _take16(rv, lanes * 0 + wlane + 1)
                    for t4 in range(4):
                        sl = pl.ds(t4 * 16, 16)
                        qrA[j, sl] = qrA[j, sl] * r0
                    for t4 in range(4, 8):
                        sl = pl.ds(t4 * 16, 16)
                        qrA[j, sl] = qrA[j, sl] * r1
                    return c2
                lax.fori_loop(0, BE, _nrow, 0)
                pltpu.sync_copy(qrA, out_h.at[pl.ds(c * NP + row0 + t * BE, BE)])
                return carry
            lax.fori_loop(0, stripe // BE, _ocp, 0)

    return body(qP, kP, vP, src, dst)


def _post_body(x_ref, agg_ref, xs_ref, g2_ref, b2_ref, w1_ref, bf1_ref,
               w2_ref, bf2_ref, rs_ref, o_ref):
    rs = rs_ref[0, 0]
    agg = jnp.concatenate([agg_ref[0], agg_ref[1]], axis=1)
    out = agg + xs_ref[...]
    x1 = x_ref[...] + rs * out
    mu = jnp.mean(x1, axis=-1, keepdims=True)
    var = jnp.mean((x1 - mu) ** 2, axis=-1, keepdims=True)
    hh = (x1 - mu) * lax.rsqrt(var + _EPS) * g2_ref[...] + b2_ref[...]
    y = jnp.dot(hh, w1_ref[...], preferred_element_type=jnp.float32) + bf1_ref[...]
    f = y * jnp.float32(0.5) * (jnp.float32(1.0) + lax.erf(y * jnp.float32(0.7071067811865476)))
    f = jnp.dot(f, w2_ref[...], preferred_element_type=jnp.float32) + bf2_ref[...]
    o_ref[...] = x1 + rs * f


def _post(x, agg, xskip, g2, beta2, W1, bf1, W2, bf2, res_scale):
    full = lambda shp: pl.BlockSpec(shp, lambda i: (0,) * len(shp))
    return pl.pallas_call(
        _post_body,
        grid=(N // BN,),
        in_specs=[
            pl.BlockSpec((BN, D), lambda i: (i, 0)),
            pl.BlockSpec((2, BN, PW), lambda i: (0, i, 0)),
            pl.BlockSpec((BN, D), lambda i: (i, 0)),
            full((1, D)), full((1, D)),
            full((D, D)), full((1, D)),
            full((D, D)), full((1, D)),
            pl.BlockSpec(memory_space=pltpu.SMEM),
        ],
        out_specs=pl.BlockSpec((BN, D), lambda i: (i, 0)),
        out_shape=jax.ShapeDtypeStruct((N, D), jnp.float32),
    )(x, agg, xskip, g2.reshape(1, D), beta2.reshape(1, D),
      W1, bf1.reshape(1, D), W2, bf2.reshape(1, D), res_scale.reshape(1, 1))


def kernel(x, edge_index, Wq, bq, Wk, bk, Wv, bv, Wskip, bskip,
           g1, beta1, g2, beta2, W1, bf1, W2, bf2, res_scale):
    src = edge_index[0]
    dst = edge_index[1]
    qP, kP, vP, xskip = _pre(x, Wq, bq, Wk, bk, Wv, bv, Wskip, bskip, g1, beta1)
    aggP, _ = _sc_edge(qP.reshape(2 * N, PW), kP.reshape(2 * N, PW),
                       vP.reshape(2 * N, PW), src, dst)
    return _post(x, aggP.reshape(2, NP, PW)[:, :N, :], xskip,
                 g2, beta2, W1, bf1, W2, bf2, res_scale)
